# baseline (speedup 1.0000x reference)
import jax
import jax.numpy as jnp
from jax.experimental import pallas as pl
from jax.experimental.pallas import tpu as pltpu
from jax.experimental.pallas import tpu_sc as plsc

_B = 4
_N = 4096
_NPOINT = 1024
_K = 16
_CIN = 128
_COUT = 256
_EPS = 1e-5

_QBLK = 256
_PBLK = 512
_P = _B * _NPOINT * _K


def _fps_body(xyz_ref, new_xyz_ref):
    x0 = xyz_ref[:, 0, :]
    x1 = xyz_ref[:, 1, :]
    x2 = xyz_ref[:, 2, :]
    iota_n = jax.lax.broadcasted_iota(jnp.int32, (_B, _N), 1)
    iota_p = jax.lax.broadcasted_iota(jnp.int32, (_B, _NPOINT), 1)

    def body(i, state):
        dists, far, ax, ay, az = state
        mask = iota_n == far
        cx = jnp.sum(jnp.where(mask, x0, 0.0), axis=1, keepdims=True)
        cy = jnp.sum(jnp.where(mask, x1, 0.0), axis=1, keepdims=True)
        cz = jnp.sum(jnp.where(mask, x2, 0.0), axis=1, keepdims=True)
        upd = iota_p == i
        ax = jnp.where(upd, cx, ax)
        ay = jnp.where(upd, cy, ay)
        az = jnp.where(upd, cz, az)
        dx = x0 - cx
        dy = x1 - cy
        dz = x2 - cz
        d = dx * dx + dy * dy
        d = d + dz * dz
        dists = jnp.minimum(dists, d)
        m = jnp.max(dists, axis=1, keepdims=True)
        far = jnp.min(jnp.where(dists == m, iota_n, _N), axis=1, keepdims=True)
        return (dists, far, ax, ay, az)

    init = (
        jnp.full((_B, _N), 1e10, dtype=jnp.float32),
        jnp.zeros((_B, 1), dtype=jnp.int32),
        jnp.zeros((_B, _NPOINT), dtype=jnp.float32),
        jnp.zeros((_B, _NPOINT), dtype=jnp.float32),
        jnp.zeros((_B, _NPOINT), dtype=jnp.float32),
    )
    _, _, ax, ay, az = jax.lax.fori_loop(0, _NPOINT, body, init)
    new_xyz_ref[:, 0, :] = ax
    new_xyz_ref[:, 1, :] = ay
    new_xyz_ref[:, 2, :] = az


def _fps(xyz):
    return pl.pallas_call(
        _fps_body,
        out_shape=jax.ShapeDtypeStruct((_B, 3, _NPOINT), jnp.float32),
    )(xyz)


def _knn_body(xyz_ref, new_xyz_ref, idx_ref):
    x0 = xyz_ref[0, 0, :][None, :]
    x1 = xyz_ref[0, 1, :][None, :]
    x2 = xyz_ref[0, 2, :][None, :]
    n0 = new_xyz_ref[0, 0, :]
    n1 = new_xyz_ref[0, 1, :]
    n2 = new_xyz_ref[0, 2, :]
    dx = n0[:, None] - x0
    dy = n1[:, None] - x1
    dz = n2[:, None] - x2
    d2 = dx * dx + dy * dy
    d2 = d2 + dz * dz
    iota_n = jax.lax.broadcasted_iota(jnp.int32, (_QBLK, _N), 1)
    for k in range(_K):
        m = jnp.min(d2, axis=1, keepdims=True)
        sel = jnp.min(jnp.where(d2 == m, iota_n, _N), axis=1, keepdims=True)
        idx_ref[0, k, :] = sel[:, 0]
        d2 = jnp.where(iota_n == sel, jnp.inf, d2)


def _knn(xyz, new_xyz):
    nqb = _NPOINT // _QBLK
    grid = (_B, nqb)
    return pl.pallas_call(
        _knn_body,
        grid=grid,
        in_specs=[
            pl.BlockSpec((1, 3, _N), lambda b, q: (b, 0, 0)),
            pl.BlockSpec((1, 3, _QBLK), lambda b, q: (b, 0, q)),
        ],
        out_specs=pl.BlockSpec((1, _K, _QBLK), lambda b, q: (b, 0, q)),
        out_shape=jax.ShapeDtypeStruct((_B, _K, _NPOINT), jnp.int32),
    )(xyz, new_xyz)


def _sc_gather(table, flat_idx, width):
    window = 128
    mesh = plsc.VectorSubcoreMesh(core_axis_name="core",
                                  subcore_axis_name="subcore")

    @pl.kernel(
        out_type=jax.ShapeDtypeStruct((_P, width), jnp.float32),
        mesh=mesh,
    )
    def kernel(x_hbm, i_hbm, o_hbm):
        def body(i_vmem, o_vmem):
            pltpu.sync_copy(x_hbm.at[i_vmem.at[0]], o_vmem)

        pltpu.emit_pipeline(
            body,
            grid=(_P // window,),
            in_specs=[pl.BlockSpec((1, window), index_map=lambda i: (0, i))],
            out_specs=[pl.BlockSpec((window, width),
                                    index_map=lambda i: (i, 0))],
            core_axis_name=("core", "subcore"),
            dimension_semantics=(pltpu.PARALLEL,),
        )(i_hbm, o_hbm)

    return kernel(table, flat_idx)


def _conv1_body(g_ref, xg_ref, nx_ref, w1b_ref, aux_ref,
                y1_ref, s1_ref, gxyz_ref):
    i = pl.program_id(0)
    gxyz = xg_ref[:, 0:3] - nx_ref[...]
    gxyz_ref[...] = gxyz
    y = jnp.dot(g_ref[...], w1b_ref[...],
                preferred_element_type=jnp.float32)
    y = y + gxyz[:, 0:1] * aux_ref[0, :][None, :]
    y = y + gxyz[:, 1:2] * aux_ref[1, :][None, :]
    y = y + gxyz[:, 2:3] * aux_ref[2, :][None, :]
    y = y + aux_ref[3, :][None, :]
    y1_ref[...] = y

    @pl.when(i == 0)
    def _():
        s1_ref[...] = jnp.zeros_like(s1_ref)

    s1_ref[0, :] += jnp.sum(y, axis=0)
    s1_ref[1, :] += jnp.sum(y * y, axis=0)


def _conv1(g, xg, nxyz_rep, w1b_t, aux1):
    grid = (_P // _PBLK,)
    return pl.pallas_call(
        _conv1_body,
        grid=grid,
        in_specs=[
            pl.BlockSpec((_PBLK, _CIN), lambda i: (i, 0)),
            pl.BlockSpec((_PBLK, _CIN), lambda i: (i, 0)),
            pl.BlockSpec((_PBLK, 3), lambda i: (i, 0)),
            pl.BlockSpec((_CIN, _CIN), lambda i: (0, 0)),
            pl.BlockSpec((8, _CIN), lambda i: (0, 0)),
        ],
        out_specs=[
            pl.BlockSpec((_PBLK, _CIN), lambda i: (i, 0)),
            pl.BlockSpec((8, _CIN), lambda i: (0, 0)),
            pl.BlockSpec((_PBLK, 3), lambda i: (i, 0)),
        ],
        out_shape=[
            jax.ShapeDtypeStruct((_P, _CIN), jnp.float32),
            jax.ShapeDtypeStruct((8, _CIN), jnp.float32),
            jax.ShapeDtypeStruct((_P, 3), jnp.float32),
        ],
    )(g, xg, nxyz_rep, w1b_t, aux1)


def _conv2_body(y1_ref, s1_ref, aux1_ref, w2_ref, aux2_ref, y2_ref, s2_ref):
    i = pl.program_id(0)
    n = jnp.float32(_P)
    mean = s1_ref[0, :] / n
    var = s1_ref[1, :] / n - mean * mean
    inv = 1.0 / jnp.sqrt(var + _EPS)
    scale = aux1_ref[0, :] * inv
    shift = aux1_ref[1, :] - mean * scale
    h = jnp.maximum(y1_ref[...] * scale[None, :] + shift[None, :], 0.0)
    y = jnp.dot(h, w2_ref[...], preferred_element_type=jnp.float32)
    y = y + aux2_ref[0, :][None, :]
    y2_ref[...] = y

    @pl.when(i == 0)
    def _():
        s2_ref[...] = jnp.zeros_like(s2_ref)

    s2_ref[0, :] += jnp.sum(y, axis=0)
    s2_ref[1, :] += jnp.sum(y * y, axis=0)


def _conv2(y1, s1, aux_gb1, w2_t, aux_b2):
    grid = (_P // _PBLK,)
    return pl.pallas_call(
        _conv2_body,
        grid=grid,
        in_specs=[
            pl.BlockSpec((_PBLK, _CIN), lambda i: (i, 0)),
            pl.BlockSpec((8, _CIN), lambda i: (0, 0)),
            pl.BlockSpec((8, _CIN), lambda i: (0, 0)),
            pl.BlockSpec((_CIN, _COUT), lambda i: (0, 0)),
            pl.BlockSpec((8, _COUT), lambda i: (0, 0)),
        ],
        out_specs=[
            pl.BlockSpec((_PBLK, _COUT), lambda i: (i, 0)),
            pl.BlockSpec((8, _COUT), lambda i: (0, 0)),
        ],
        out_shape=[
            jax.ShapeDtypeStruct((_P, _COUT), jnp.float32),
            jax.ShapeDtypeStruct((8, _COUT), jnp.float32),
        ],
    )(y1, s1, aux_gb1, w2_t, aux_b2)


def _final_body(y2_ref, s2_ref, aux2_ref, np_ref, pool_ref):
    n = jnp.float32(_P)
    mean = s2_ref[0, :] / n
    var = s2_ref[1, :] / n - mean * mean
    inv = 1.0 / jnp.sqrt(var + _EPS)
    scale = aux2_ref[0, :] * inv
    shift = aux2_ref[1, :] - mean * scale
    o = jnp.maximum(y2_ref[...] * scale[None, :] + shift[None, :], 0.0)
    np_ref[0] = o.T
    pool_ref[...] = jnp.max(o.reshape(_PBLK // _K, _K, _COUT), axis=1)


def _finalize(y2, s2, aux_gb2):
    grid = (_P // _PBLK,)
    nqb = (_NPOINT * _K) // _PBLK
    qblk = _PBLK // _K
    return pl.pallas_call(
        _final_body,
        grid=grid,
        in_specs=[
            pl.BlockSpec((_PBLK, _COUT), lambda i: (i, 0)),
            pl.BlockSpec((8, _COUT), lambda i: (0, 0)),
            pl.BlockSpec((8, _COUT), lambda i: (0, 0)),
        ],
        out_specs=[
            pl.BlockSpec((1, _COUT, _PBLK), lambda i: (i // nqb, 0, i % nqb)),
            pl.BlockSpec((qblk, _COUT), lambda i: (i, 0)),
        ],
        out_shape=[
            jax.ShapeDtypeStruct((_B, _COUT, _NPOINT * _K), jnp.float32),
            jax.ShapeDtypeStruct((_B * _NPOINT, _COUT), jnp.float32),
        ],
    )(y2, s2, aux_gb2)


def kernel(xyz, points, W1, b1, gamma1, beta1, W2, b2, gamma2, beta2):
    new_xyz = _fps(xyz)
    idx_kn = _knn(xyz, new_xyz)

    points_pm = jnp.transpose(points, (0, 2, 1)).reshape(_B * _N, _CIN)
    xyz_pm = jnp.transpose(xyz, (0, 2, 1)).reshape(_B * _N, 3)
    xyz_pad = jnp.pad(xyz_pm, ((0, 0), (0, 125)))
    idx = jnp.transpose(idx_kn, (0, 2, 1))
    offs = (jnp.arange(_B, dtype=jnp.int32) * _N)[:, None]
    flat_idx = (idx.reshape(_B, -1) + offs).reshape(1, _P)
    g = _sc_gather(points_pm, flat_idx, _CIN)
    xg = _sc_gather(xyz_pad, flat_idx, 128)

    nxyz_t = jnp.transpose(new_xyz, (0, 2, 1))
    nxyz_rep = jnp.broadcast_to(
        nxyz_t[:, :, None, :], (_B, _NPOINT, _K, 3)).reshape(_P, 3)

    zpad = jnp.zeros((4, _CIN), jnp.float32)
    aux1 = jnp.concatenate([W1[:, :3].T, b1[None, :], zpad], axis=0)
    w1b_t = W1[:, 3:].T
    y1, s1, gxyz_pm = _conv1(g, xg, nxyz_rep, w1b_t, aux1)
    gxyz = jnp.transpose(
        gxyz_pm.reshape(_B, _NPOINT, _K, 3), (0, 3, 1, 2))

    zpad1 = jnp.zeros((6, _CIN), jnp.float32)
    aux_gb1 = jnp.concatenate([gamma1[None, :], beta1[None, :], zpad1], axis=0)
    zpad2 = jnp.zeros((7, _COUT), jnp.float32)
    aux_b2 = jnp.concatenate([b2[None, :], zpad2], axis=0)
    y2, s2 = _conv2(y1, s1, aux_gb1, W2.T, aux_b2)

    zpad3 = jnp.zeros((6, _COUT), jnp.float32)
    aux_gb2 = jnp.concatenate([gamma2[None, :], beta2[None, :], zpad3], axis=0)
    np_cm, pool_pm = _finalize(y2, s2, aux_gb2)

    new_points = np_cm.reshape(_B, _COUT, _NPOINT, _K)
    pooled = jnp.transpose(pool_pm.reshape(_B, _NPOINT, _COUT), (0, 2, 1))
    return (new_xyz, pooled, gxyz, new_points)


# speedup vs baseline: 1.0884x; 1.0884x over previous
import jax
import jax.numpy as jnp
from jax.experimental import pallas as pl
from jax.experimental.pallas import tpu as pltpu
from jax.experimental.pallas import tpu_sc as plsc

_B = 4
_N = 4096
_NPOINT = 1024
_K = 16
_CIN = 128
_COUT = 256
_EPS = 1e-5

_QBLK = 256
_PBLK = 512
_P = _B * _NPOINT * _K


_FS = 8
_FL = _N // _FS
_PL = _NPOINT // _FS


def _rmin(x):
    return jnp.min(jnp.min(x, axis=2, keepdims=True), axis=1, keepdims=True)


def _rmax(x):
    return jnp.max(jnp.max(x, axis=2, keepdims=True), axis=1, keepdims=True)


def _rsum(x):
    return jnp.sum(jnp.sum(x, axis=2, keepdims=True), axis=1, keepdims=True)


def _fps_body(xyz_ref, new_xyz_ref):
    x0 = xyz_ref[:, 0]
    x1 = xyz_ref[:, 1]
    x2 = xyz_ref[:, 2]
    sh = (_B, _FS, _FL)
    iota_n = (jax.lax.broadcasted_iota(jnp.int32, sh, 1) * _FL
              + jax.lax.broadcasted_iota(jnp.int32, sh, 2))
    shp = (_B, _FS, _PL)
    iota_p = (jax.lax.broadcasted_iota(jnp.int32, shp, 1) * _PL
              + jax.lax.broadcasted_iota(jnp.int32, shp, 2))

    def body(i, state):
        dists, far, ax, ay, az = state
        mask = iota_n == far
        cx = _rsum(jnp.where(mask, x0, 0.0))
        cy = _rsum(jnp.where(mask, x1, 0.0))
        cz = _rsum(jnp.where(mask, x2, 0.0))
        upd = iota_p == i
        ax = jnp.where(upd, cx, ax)
        ay = jnp.where(upd, cy, ay)
        az = jnp.where(upd, cz, az)
        dx = x0 - cx
        dy = x1 - cy
        dz = x2 - cz
        d = dx * dx + dy * dy
        d = d + dz * dz
        dists = jnp.minimum(dists, d)
        m = _rmax(dists)
        far = _rmin(jnp.where(dists == m, iota_n, _N))
        return (dists, far, ax, ay, az)

    init = (
        jnp.full((_B, _FS, _FL), 1e10, dtype=jnp.float32),
        jnp.zeros((_B, 1, 1), dtype=jnp.int32),
        jnp.zeros((_B, _FS, _PL), dtype=jnp.float32),
        jnp.zeros((_B, _FS, _PL), dtype=jnp.float32),
        jnp.zeros((_B, _FS, _PL), dtype=jnp.float32),
    )
    _, _, ax, ay, az = jax.lax.fori_loop(0, _NPOINT, body, init)
    new_xyz_ref[:, 0] = ax
    new_xyz_ref[:, 1] = ay
    new_xyz_ref[:, 2] = az


def _fps(xyz):
    out = pl.pallas_call(
        _fps_body,
        out_shape=jax.ShapeDtypeStruct((_B, 3, _FS, _PL), jnp.float32),
    )(xyz.reshape(_B, 3, _FS, _FL))
    return out.reshape(_B, 3, _NPOINT)


def _knn_body(xyz_ref, new_xyz_ref, idx_ref):
    x0 = xyz_ref[0, 0, :][None, :]
    x1 = xyz_ref[0, 1, :][None, :]
    x2 = xyz_ref[0, 2, :][None, :]
    n0 = new_xyz_ref[0, 0, :]
    n1 = new_xyz_ref[0, 1, :]
    n2 = new_xyz_ref[0, 2, :]
    dx = n0[:, None] - x0
    dy = n1[:, None] - x1
    dz = n2[:, None] - x2
    d2 = dx * dx + dy * dy
    d2 = d2 + dz * dz
    iota_n = jax.lax.broadcasted_iota(jnp.int32, (_QBLK, _N), 1)
    for k in range(_K):
        m = jnp.min(d2, axis=1, keepdims=True)
        sel = jnp.min(jnp.where(d2 == m, iota_n, _N), axis=1, keepdims=True)
        idx_ref[0, k, :] = sel[:, 0]
        d2 = jnp.where(iota_n == sel, jnp.inf, d2)


def _knn(xyz, new_xyz):
    nqb = _NPOINT // _QBLK
    grid = (_B, nqb)
    return pl.pallas_call(
        _knn_body,
        grid=grid,
        in_specs=[
            pl.BlockSpec((1, 3, _N), lambda b, q: (b, 0, 0)),
            pl.BlockSpec((1, 3, _QBLK), lambda b, q: (b, 0, q)),
        ],
        out_specs=pl.BlockSpec((1, _K, _QBLK), lambda b, q: (b, 0, q)),
        out_shape=jax.ShapeDtypeStruct((_B, _K, _NPOINT), jnp.int32),
    )(xyz, new_xyz)


def _sc_gather(table, flat_idx, width):
    window = 128
    mesh = plsc.VectorSubcoreMesh(core_axis_name="core",
                                  subcore_axis_name="subcore")

    @pl.kernel(
        out_type=jax.ShapeDtypeStruct((_P, width), jnp.float32),
        mesh=mesh,
    )
    def kernel(x_hbm, i_hbm, o_hbm):
        def body(i_vmem, o_vmem):
            pltpu.sync_copy(x_hbm.at[i_vmem.at[0]], o_vmem)

        pltpu.emit_pipeline(
            body,
            grid=(_P // window,),
            in_specs=[pl.BlockSpec((1, window), index_map=lambda i: (0, i))],
            out_specs=[pl.BlockSpec((window, width),
                                    index_map=lambda i: (i, 0))],
            core_axis_name=("core", "subcore"),
            dimension_semantics=(pltpu.PARALLEL,),
        )(i_hbm, o_hbm)

    return kernel(table, flat_idx)


def _conv1_body(g_ref, xg_ref, nx_ref, w1b_ref, aux_ref,
                y1_ref, s1_ref, gxyz_ref):
    i = pl.program_id(0)
    gxyz = xg_ref[:, 0:3] - nx_ref[...]
    gxyz_ref[...] = gxyz
    y = jnp.dot(g_ref[...], w1b_ref[...],
                preferred_element_type=jnp.float32)
    y = y + gxyz[:, 0:1] * aux_ref[0, :][None, :]
    y = y + gxyz[:, 1:2] * aux_ref[1, :][None, :]
    y = y + gxyz[:, 2:3] * aux_ref[2, :][None, :]
    y = y + aux_ref[3, :][None, :]
    y1_ref[...] = y

    @pl.when(i == 0)
    def _():
        s1_ref[...] = jnp.zeros_like(s1_ref)

    s1_ref[0, :] += jnp.sum(y, axis=0)
    s1_ref[1, :] += jnp.sum(y * y, axis=0)


def _conv1(g, xg, nxyz_rep, w1b_t, aux1):
    grid = (_P // _PBLK,)
    return pl.pallas_call(
        _conv1_body,
        grid=grid,
        in_specs=[
            pl.BlockSpec((_PBLK, _CIN), lambda i: (i, 0)),
            pl.BlockSpec((_PBLK, _CIN), lambda i: (i, 0)),
            pl.BlockSpec((_PBLK, 3), lambda i: (i, 0)),
            pl.BlockSpec((_CIN, _CIN), lambda i: (0, 0)),
            pl.BlockSpec((8, _CIN), lambda i: (0, 0)),
        ],
        out_specs=[
            pl.BlockSpec((_PBLK, _CIN), lambda i: (i, 0)),
            pl.BlockSpec((8, _CIN), lambda i: (0, 0)),
            pl.BlockSpec((_PBLK, 3), lambda i: (i, 0)),
        ],
        out_shape=[
            jax.ShapeDtypeStruct((_P, _CIN), jnp.float32),
            jax.ShapeDtypeStruct((8, _CIN), jnp.float32),
            jax.ShapeDtypeStruct((_P, 3), jnp.float32),
        ],
    )(g, xg, nxyz_rep, w1b_t, aux1)


def _conv2_body(y1_ref, s1_ref, aux1_ref, w2_ref, aux2_ref, y2_ref, s2_ref):
    i = pl.program_id(0)
    n = jnp.float32(_P)
    mean = s1_ref[0, :] / n
    var = s1_ref[1, :] / n - mean * mean
    inv = 1.0 / jnp.sqrt(var + _EPS)
    scale = aux1_ref[0, :] * inv
    shift = aux1_ref[1, :] - mean * scale
    h = jnp.maximum(y1_ref[...] * scale[None, :] + shift[None, :], 0.0)
    y = jnp.dot(h, w2_ref[...], preferred_element_type=jnp.float32)
    y = y + aux2_ref[0, :][None, :]
    y2_ref[...] = y

    @pl.when(i == 0)
    def _():
        s2_ref[...] = jnp.zeros_like(s2_ref)

    s2_ref[0, :] += jnp.sum(y, axis=0)
    s2_ref[1, :] += jnp.sum(y * y, axis=0)


def _conv2(y1, s1, aux_gb1, w2_t, aux_b2):
    grid = (_P // _PBLK,)
    return pl.pallas_call(
        _conv2_body,
        grid=grid,
        in_specs=[
            pl.BlockSpec((_PBLK, _CIN), lambda i: (i, 0)),
            pl.BlockSpec((8, _CIN), lambda i: (0, 0)),
            pl.BlockSpec((8, _CIN), lambda i: (0, 0)),
            pl.BlockSpec((_CIN, _COUT), lambda i: (0, 0)),
            pl.BlockSpec((8, _COUT), lambda i: (0, 0)),
        ],
        out_specs=[
            pl.BlockSpec((_PBLK, _COUT), lambda i: (i, 0)),
            pl.BlockSpec((8, _COUT), lambda i: (0, 0)),
        ],
        out_shape=[
            jax.ShapeDtypeStruct((_P, _COUT), jnp.float32),
            jax.ShapeDtypeStruct((8, _COUT), jnp.float32),
        ],
    )(y1, s1, aux_gb1, w2_t, aux_b2)


def _final_body(y2_ref, s2_ref, aux2_ref, np_ref, pool_ref):
    n = jnp.float32(_P)
    mean = s2_ref[0, :] / n
    var = s2_ref[1, :] / n - mean * mean
    inv = 1.0 / jnp.sqrt(var + _EPS)
    scale = aux2_ref[0, :] * inv
    shift = aux2_ref[1, :] - mean * scale
    o = jnp.maximum(y2_ref[...] * scale[None, :] + shift[None, :], 0.0)
    np_ref[0] = o.T
    pool_ref[...] = jnp.max(o.reshape(_PBLK // _K, _K, _COUT), axis=1)


def _finalize(y2, s2, aux_gb2):
    grid = (_P // _PBLK,)
    nqb = (_NPOINT * _K) // _PBLK
    qblk = _PBLK // _K
    return pl.pallas_call(
        _final_body,
        grid=grid,
        in_specs=[
            pl.BlockSpec((_PBLK, _COUT), lambda i: (i, 0)),
            pl.BlockSpec((8, _COUT), lambda i: (0, 0)),
            pl.BlockSpec((8, _COUT), lambda i: (0, 0)),
        ],
        out_specs=[
            pl.BlockSpec((1, _COUT, _PBLK), lambda i: (i // nqb, 0, i % nqb)),
            pl.BlockSpec((qblk, _COUT), lambda i: (i, 0)),
        ],
        out_shape=[
            jax.ShapeDtypeStruct((_B, _COUT, _NPOINT * _K), jnp.float32),
            jax.ShapeDtypeStruct((_B * _NPOINT, _COUT), jnp.float32),
        ],
    )(y2, s2, aux_gb2)


def kernel(xyz, points, W1, b1, gamma1, beta1, W2, b2, gamma2, beta2):
    new_xyz = _fps(xyz)
    idx_kn = _knn(xyz, new_xyz)

    points_pm = jnp.transpose(points, (0, 2, 1)).reshape(_B * _N, _CIN)
    xyz_pm = jnp.transpose(xyz, (0, 2, 1)).reshape(_B * _N, 3)
    xyz_pad = jnp.pad(xyz_pm, ((0, 0), (0, 125)))
    idx = jnp.transpose(idx_kn, (0, 2, 1))
    offs = (jnp.arange(_B, dtype=jnp.int32) * _N)[:, None]
    flat_idx = (idx.reshape(_B, -1) + offs).reshape(1, _P)
    g = _sc_gather(points_pm, flat_idx, _CIN)
    xg = _sc_gather(xyz_pad, flat_idx, 128)

    nxyz_t = jnp.transpose(new_xyz, (0, 2, 1))
    nxyz_rep = jnp.broadcast_to(
        nxyz_t[:, :, None, :], (_B, _NPOINT, _K, 3)).reshape(_P, 3)

    zpad = jnp.zeros((4, _CIN), jnp.float32)
    aux1 = jnp.concatenate([W1[:, :3].T, b1[None, :], zpad], axis=0)
    w1b_t = W1[:, 3:].T
    y1, s1, gxyz_pm = _conv1(g, xg, nxyz_rep, w1b_t, aux1)
    gxyz = jnp.transpose(
        gxyz_pm.reshape(_B, _NPOINT, _K, 3), (0, 3, 1, 2))

    zpad1 = jnp.zeros((6, _CIN), jnp.float32)
    aux_gb1 = jnp.concatenate([gamma1[None, :], beta1[None, :], zpad1], axis=0)
    zpad2 = jnp.zeros((7, _COUT), jnp.float32)
    aux_b2 = jnp.concatenate([b2[None, :], zpad2], axis=0)
    y2, s2 = _conv2(y1, s1, aux_gb1, W2.T, aux_b2)

    zpad3 = jnp.zeros((6, _COUT), jnp.float32)
    aux_gb2 = jnp.concatenate([gamma2[None, :], beta2[None, :], zpad3], axis=0)
    np_cm, pool_pm = _finalize(y2, s2, aux_gb2)

    new_points = np_cm.reshape(_B, _COUT, _NPOINT, _K)
    pooled = jnp.transpose(pool_pm.reshape(_B, _NPOINT, _COUT), (0, 2, 1))
    return (new_xyz, pooled, gxyz, new_points)


# speedup vs baseline: 1.0996x; 1.0102x over previous
import jax
import jax.numpy as jnp
from jax.experimental import pallas as pl
from jax.experimental.pallas import tpu as pltpu
from jax.experimental.pallas import tpu_sc as plsc

_B = 4
_N = 4096
_NPOINT = 1024
_K = 16
_CIN = 128
_COUT = 256
_EPS = 1e-5

_QBLK = 256
_PBLK = 512
_P = _B * _NPOINT * _K


_FS = 8
_FL = _N // _FS
_PL = _NPOINT // _FS


def _rmin(x):
    return jnp.min(jnp.min(x, axis=2, keepdims=True), axis=1, keepdims=True)


def _rmax(x):
    return jnp.max(jnp.max(x, axis=2, keepdims=True), axis=1, keepdims=True)


def _rsum(x):
    return jnp.sum(jnp.sum(x, axis=2, keepdims=True), axis=1, keepdims=True)


def _fps_body(xyz_ref, new_xyz_ref):
    x0 = xyz_ref[:, 0]
    x1 = xyz_ref[:, 1]
    x2 = xyz_ref[:, 2]
    sh = (_B, _FS, _FL)
    iota_n = (jax.lax.broadcasted_iota(jnp.int32, sh, 1) * _FL
              + jax.lax.broadcasted_iota(jnp.int32, sh, 2))
    shp = (_B, _FS, _PL)
    iota_p = (jax.lax.broadcasted_iota(jnp.int32, shp, 1) * _PL
              + jax.lax.broadcasted_iota(jnp.int32, shp, 2))

    def body(i, state):
        dists, far, ax, ay, az = state
        mask = iota_n == far
        cx = _rsum(jnp.where(mask, x0, 0.0))
        cy = _rsum(jnp.where(mask, x1, 0.0))
        cz = _rsum(jnp.where(mask, x2, 0.0))
        upd = iota_p == i
        ax = jnp.where(upd, cx, ax)
        ay = jnp.where(upd, cy, ay)
        az = jnp.where(upd, cz, az)
        dx = x0 - cx
        dy = x1 - cy
        dz = x2 - cz
        d = dx * dx + dy * dy
        d = d + dz * dz
        dists = jnp.minimum(dists, d)
        m = _rmax(dists)
        far = _rmin(jnp.where(dists == m, iota_n, _N))
        return (dists, far, ax, ay, az)

    init = (
        jnp.full((_B, _FS, _FL), 1e10, dtype=jnp.float32),
        jnp.zeros((_B, 1, 1), dtype=jnp.int32),
        jnp.zeros((_B, _FS, _PL), dtype=jnp.float32),
        jnp.zeros((_B, _FS, _PL), dtype=jnp.float32),
        jnp.zeros((_B, _FS, _PL), dtype=jnp.float32),
    )
    _, _, ax, ay, az = jax.lax.fori_loop(0, _NPOINT, body, init)
    new_xyz_ref[:, 0] = ax
    new_xyz_ref[:, 1] = ay
    new_xyz_ref[:, 2] = az


def _fps(xyz):
    out = pl.pallas_call(
        _fps_body,
        out_shape=jax.ShapeDtypeStruct((_B, 3, _FS, _PL), jnp.float32),
    )(xyz.reshape(_B, 3, _FS, _FL))
    return out.reshape(_B, 3, _NPOINT)


def _knn_body(xyz_ref, new_xyz_ref, idx_ref):
    x0 = xyz_ref[0, 0, :][None, :]
    x1 = xyz_ref[0, 1, :][None, :]
    x2 = xyz_ref[0, 2, :][None, :]
    n0 = new_xyz_ref[0, 0, :]
    n1 = new_xyz_ref[0, 1, :]
    n2 = new_xyz_ref[0, 2, :]
    dx = n0[:, None] - x0
    dy = n1[:, None] - x1
    dz = n2[:, None] - x2
    d2 = dx * dx + dy * dy
    d2 = d2 + dz * dz
    iota_n = jax.lax.broadcasted_iota(jnp.int32, (_QBLK, _N), 1)
    for k in range(_K):
        m = jnp.min(d2, axis=1, keepdims=True)
        sel = jnp.min(jnp.where(d2 == m, iota_n, _N), axis=1, keepdims=True)
        idx_ref[0, k, :] = sel[:, 0]
        d2 = jnp.where(iota_n == sel, jnp.inf, d2)


def _knn(xyz, new_xyz):
    nqb = _NPOINT // _QBLK
    grid = (_B, nqb)
    return pl.pallas_call(
        _knn_body,
        grid=grid,
        in_specs=[
            pl.BlockSpec((1, 3, _N), lambda b, q: (b, 0, 0)),
            pl.BlockSpec((1, 3, _QBLK), lambda b, q: (b, 0, q)),
        ],
        out_specs=pl.BlockSpec((1, _K, _QBLK), lambda b, q: (b, 0, q)),
        out_shape=jax.ShapeDtypeStruct((_B, _K, _NPOINT), jnp.int32),
    )(xyz, new_xyz)


def _sc_gather2(tab_a, tab_b, flat_idx):
    window = 128
    mesh = plsc.VectorSubcoreMesh(core_axis_name="core",
                                  subcore_axis_name="subcore")

    @pl.kernel(
        out_type=[jax.ShapeDtypeStruct((_P, _CIN), jnp.float32),
                  jax.ShapeDtypeStruct((_P, _CIN), jnp.float32)],
        mesh=mesh,
    )
    def kernel(a_hbm, b_hbm, i_hbm, oa_hbm, ob_hbm):
        def body(i_vmem, oa_vmem, ob_vmem):
            pltpu.sync_copy(a_hbm.at[i_vmem.at[0]], oa_vmem)
            pltpu.sync_copy(b_hbm.at[i_vmem.at[0]], ob_vmem)

        pltpu.emit_pipeline(
            body,
            grid=(_P // window,),
            in_specs=[pl.BlockSpec((1, window), index_map=lambda i: (0, i))],
            out_specs=[pl.BlockSpec((window, _CIN),
                                    index_map=lambda i: (i, 0)),
                       pl.BlockSpec((window, _CIN),
                                    index_map=lambda i: (i, 0))],
            core_axis_name=("core", "subcore"),
            dimension_semantics=(pltpu.PARALLEL,),
        )(i_hbm, oa_hbm, ob_hbm)

    return kernel(tab_a, tab_b, flat_idx)


def _conv1_body(g_ref, xg_ref, nx_ref, w1b_ref, aux_ref,
                y1_ref, s1_ref, gxyz_ref):
    i = pl.program_id(0)
    gxyz = xg_ref[:, 0:3] - nx_ref[...]
    gxyz_ref[...] = gxyz
    y = jnp.dot(g_ref[...], w1b_ref[...],
                preferred_element_type=jnp.float32)
    y = y + gxyz[:, 0:1] * aux_ref[0, :][None, :]
    y = y + gxyz[:, 1:2] * aux_ref[1, :][None, :]
    y = y + gxyz[:, 2:3] * aux_ref[2, :][None, :]
    y = y + aux_ref[3, :][None, :]
    y1_ref[...] = y

    @pl.when(i == 0)
    def _():
        s1_ref[...] = jnp.zeros_like(s1_ref)

    s1_ref[0, :] += jnp.sum(y, axis=0)
    s1_ref[1, :] += jnp.sum(y * y, axis=0)


def _conv1(g, xg, nxyz_rep, w1b_t, aux1):
    grid = (_P // _PBLK,)
    return pl.pallas_call(
        _conv1_body,
        grid=grid,
        in_specs=[
            pl.BlockSpec((_PBLK, _CIN), lambda i: (i, 0)),
            pl.BlockSpec((_PBLK, _CIN), lambda i: (i, 0)),
            pl.BlockSpec((_PBLK, 3), lambda i: (i, 0)),
            pl.BlockSpec((_CIN, _CIN), lambda i: (0, 0)),
            pl.BlockSpec((8, _CIN), lambda i: (0, 0)),
        ],
        out_specs=[
            pl.BlockSpec((_PBLK, _CIN), lambda i: (i, 0)),
            pl.BlockSpec((8, _CIN), lambda i: (0, 0)),
            pl.BlockSpec((_PBLK, 3), lambda i: (i, 0)),
        ],
        out_shape=[
            jax.ShapeDtypeStruct((_P, _CIN), jnp.float32),
            jax.ShapeDtypeStruct((8, _CIN), jnp.float32),
            jax.ShapeDtypeStruct((_P, 3), jnp.float32),
        ],
    )(g, xg, nxyz_rep, w1b_t, aux1)


def _conv2_body(y1_ref, s1_ref, aux1_ref, w2_ref, aux2_ref, y2_ref, s2_ref):
    i = pl.program_id(0)
    n = jnp.float32(_P)
    mean = s1_ref[0, :] / n
    var = s1_ref[1, :] / n - mean * mean
    inv = 1.0 / jnp.sqrt(var + _EPS)
    scale = aux1_ref[0, :] * inv
    shift = aux1_ref[1, :] - mean * scale
    h = jnp.maximum(y1_ref[...] * scale[None, :] + shift[None, :], 0.0)
    y = jnp.dot(h, w2_ref[...], preferred_element_type=jnp.float32)
    y = y + aux2_ref[0, :][None, :]
    y2_ref[...] = y

    @pl.when(i == 0)
    def _():
        s2_ref[...] = jnp.zeros_like(s2_ref)

    s2_ref[0, :] += jnp.sum(y, axis=0)
    s2_ref[1, :] += jnp.sum(y * y, axis=0)


def _conv2(y1, s1, aux_gb1, w2_t, aux_b2):
    grid = (_P // _PBLK,)
    return pl.pallas_call(
        _conv2_body,
        grid=grid,
        in_specs=[
            pl.BlockSpec((_PBLK, _CIN), lambda i: (i, 0)),
            pl.BlockSpec((8, _CIN), lambda i: (0, 0)),
            pl.BlockSpec((8, _CIN), lambda i: (0, 0)),
            pl.BlockSpec((_CIN, _COUT), lambda i: (0, 0)),
            pl.BlockSpec((8, _COUT), lambda i: (0, 0)),
        ],
        out_specs=[
            pl.BlockSpec((_PBLK, _COUT), lambda i: (i, 0)),
            pl.BlockSpec((8, _COUT), lambda i: (0, 0)),
        ],
        out_shape=[
            jax.ShapeDtypeStruct((_P, _COUT), jnp.float32),
            jax.ShapeDtypeStruct((8, _COUT), jnp.float32),
        ],
    )(y1, s1, aux_gb1, w2_t, aux_b2)


def _final_body(y2_ref, s2_ref, aux2_ref, np_ref, pool_ref):
    n = jnp.float32(_P)
    mean = s2_ref[0, :] / n
    var = s2_ref[1, :] / n - mean * mean
    inv = 1.0 / jnp.sqrt(var + _EPS)
    scale = aux2_ref[0, :] * inv
    shift = aux2_ref[1, :] - mean * scale
    o = jnp.maximum(y2_ref[...] * scale[None, :] + shift[None, :], 0.0)
    np_ref[0] = o.T
    pool_ref[...] = jnp.max(o.reshape(_PBLK // _K, _K, _COUT), axis=1)


def _finalize(y2, s2, aux_gb2):
    grid = (_P // _PBLK,)
    nqb = (_NPOINT * _K) // _PBLK
    qblk = _PBLK // _K
    return pl.pallas_call(
        _final_body,
        grid=grid,
        in_specs=[
            pl.BlockSpec((_PBLK, _COUT), lambda i: (i, 0)),
            pl.BlockSpec((8, _COUT), lambda i: (0, 0)),
            pl.BlockSpec((8, _COUT), lambda i: (0, 0)),
        ],
        out_specs=[
            pl.BlockSpec((1, _COUT, _PBLK), lambda i: (i // nqb, 0, i % nqb)),
            pl.BlockSpec((qblk, _COUT), lambda i: (i, 0)),
        ],
        out_shape=[
            jax.ShapeDtypeStruct((_B, _COUT, _NPOINT * _K), jnp.float32),
            jax.ShapeDtypeStruct((_B * _NPOINT, _COUT), jnp.float32),
        ],
    )(y2, s2, aux_gb2)


def kernel(xyz, points, W1, b1, gamma1, beta1, W2, b2, gamma2, beta2):
    new_xyz = _fps(xyz)
    idx_kn = _knn(xyz, new_xyz)

    points_pm = jnp.transpose(points, (0, 2, 1)).reshape(_B * _N, _CIN)
    xyz_pm = jnp.transpose(xyz, (0, 2, 1)).reshape(_B * _N, 3)
    xyz_pad = jnp.pad(xyz_pm, ((0, 0), (0, 125)))
    idx = jnp.transpose(idx_kn, (0, 2, 1))
    offs = (jnp.arange(_B, dtype=jnp.int32) * _N)[:, None]
    flat_idx = (idx.reshape(_B, -1) + offs).reshape(1, _P)
    g, xg = _sc_gather2(points_pm, xyz_pad, flat_idx)

    nxyz_t = jnp.transpose(new_xyz, (0, 2, 1))
    nxyz_rep = jnp.broadcast_to(
        nxyz_t[:, :, None, :], (_B, _NPOINT, _K, 3)).reshape(_P, 3)

    zpad = jnp.zeros((4, _CIN), jnp.float32)
    aux1 = jnp.concatenate([W1[:, :3].T, b1[None, :], zpad], axis=0)
    w1b_t = W1[:, 3:].T
    y1, s1, gxyz_pm = _conv1(g, xg, nxyz_rep, w1b_t, aux1)
    gxyz = jnp.transpose(
        gxyz_pm.reshape(_B, _NPOINT, _K, 3), (0, 3, 1, 2))

    zpad1 = jnp.zeros((6, _CIN), jnp.float32)
    aux_gb1 = jnp.concatenate([gamma1[None, :], beta1[None, :], zpad1], axis=0)
    zpad2 = jnp.zeros((7, _COUT), jnp.float32)
    aux_b2 = jnp.concatenate([b2[None, :], zpad2], axis=0)
    y2, s2 = _conv2(y1, s1, aux_gb1, W2.T, aux_b2)

    zpad3 = jnp.zeros((6, _COUT), jnp.float32)
    aux_gb2 = jnp.concatenate([gamma2[None, :], beta2[None, :], zpad3], axis=0)
    np_cm, pool_pm = _finalize(y2, s2, aux_gb2)

    new_points = np_cm.reshape(_B, _COUT, _NPOINT, _K)
    pooled = jnp.transpose(pool_pm.reshape(_B, _NPOINT, _COUT), (0, 2, 1))
    return (new_xyz, pooled, gxyz, new_points)
